# Initial kernel scaffold; baseline (speedup 1.0000x reference)
import jax
import jax.numpy as jnp
from jax.experimental import pallas as pl


def kernel(feat, edge_index, W_self, W_neigh, b_neigh):
    raise NotImplementedError("write your pallas kernel here")


# speedup vs baseline: 4.0854x; 4.0854x over previous
import functools

import jax
import jax.numpy as jnp
from jax import lax
from jax.experimental import pallas as pl
from jax.experimental.pallas import tpu as pltpu
from jax.experimental.pallas import tpu_sc as plsc

N_NODES = 10000
D_IN = 128
D_OUT = 128
N_EDGES = 320000

DE = 144
NC = 2
NS = 16
NW = NC * NS
CHUNK = 128
CH_PER_W = 79
E_PER_W = CH_PER_W * CHUNK
E_PAD = NW * E_PER_W
ACC_ROWS = 10240
DUMP_ROW = N_NODES
ROWS_PER_TILE = ACC_ROWS // NS

_sc_mesh = plsc.VectorSubcoreMesh(
    core_axis_name="c", subcore_axis_name="s", num_cores=NC, num_subcores=NS)


@functools.partial(
    pl.kernel,
    out_type=jax.ShapeDtypeStruct((NC, ACC_ROWS, DE), jnp.float32),
    mesh=_sc_mesh,
    compiler_params=pltpu.CompilerParams(use_tc_tiling_on_sc=False),
    scratch_types=[
        pltpu.VMEM((CHUNK,), jnp.int32),
        pltpu.VMEM((CHUNK,), jnp.int32),
        pltpu.VMEM((CHUNK, DE), jnp.float32),
        pltpu.VMEM_SHARED((ACC_ROWS, DE), jnp.float32),
    ],
)
def _sc_aggregate(featext_hbm, src_hbm, dst_hbm, zeros_hbm, out_hbm,
                  src_v, dst_v, rows_v, acc_sh):
    c = lax.axis_index("c")
    s = lax.axis_index("s")
    wid = s * NC + c

    pltpu.sync_copy(zeros_hbm, acc_sh.at[pl.ds(s * ROWS_PER_TILE, ROWS_PER_TILE)])
    plsc.subcore_barrier()

    def body(j, carry):
        base = wid * E_PER_W + j * CHUNK
        pltpu.sync_copy(src_hbm.at[pl.ds(base, CHUNK)], src_v)
        pltpu.sync_copy(dst_hbm.at[pl.ds(base, CHUNK)], dst_v)
        pltpu.sync_copy(featext_hbm.at[src_v], rows_v)
        pltpu.sync_copy(rows_v, acc_sh.at[dst_v], add=True)
        return carry

    lax.fori_loop(0, CH_PER_W, body, 0)
    plsc.subcore_barrier()

    pltpu.sync_copy(acc_sh.at[pl.ds(s * ROWS_PER_TILE, ROWS_PER_TILE)],
                    out_hbm.at[c, pl.ds(s * ROWS_PER_TILE, ROWS_PER_TILE)])


def _tc_combine_body(x_ref, p0_ref, p1_ref, ws_ref, wn_ref, b_ref, o_ref):
    x = x_ref[...]
    p = p0_ref[...] + p1_ref[...]
    neigh_sum = p[:, :D_IN]
    deg = p[:, D_IN:D_IN + 1]
    h_neigh = neigh_sum / jnp.maximum(deg, 1.0)
    dn = (((1,), (1,)), ((), ()))
    out = lax.dot_general(x, ws_ref[...], dn, preferred_element_type=jnp.float32)
    out += lax.dot_general(h_neigh, wn_ref[...], dn, preferred_element_type=jnp.float32)
    o_ref[...] = out + b_ref[...]


def _tc_combine(feat, p0, p1, w_self, w_neigh, b2):
    blk = 1000
    grid = N_NODES // blk
    return pl.pallas_call(
        _tc_combine_body,
        grid=(grid,),
        in_specs=[
            pl.BlockSpec((blk, D_IN), lambda i: (i, 0)),
            pl.BlockSpec((blk, DE), lambda i: (i, 0)),
            pl.BlockSpec((blk, DE), lambda i: (i, 0)),
            pl.BlockSpec((D_OUT, D_IN), lambda i: (0, 0)),
            pl.BlockSpec((D_OUT, D_IN), lambda i: (0, 0)),
            pl.BlockSpec((1, D_OUT), lambda i: (0, 0)),
        ],
        out_specs=pl.BlockSpec((blk, D_OUT), lambda i: (i, 0)),
        out_shape=jax.ShapeDtypeStruct((N_NODES, D_OUT), jnp.float32),
    )(feat, p0, p1, w_self, w_neigh, b2)


def kernel(feat, edge_index, W_self, W_neigh, b_neigh):
    ones = jnp.ones((N_NODES, 1), jnp.float32)
    pad_cols = jnp.zeros((N_NODES, DE - D_IN - 1), jnp.float32)
    feat_ext = jnp.concatenate([feat, ones, pad_cols], axis=1)

    n_pad = E_PAD - N_EDGES
    src_p = jnp.concatenate([edge_index[0], jnp.zeros((n_pad,), jnp.int32)])
    dst_p = jnp.concatenate([edge_index[1],
                             jnp.full((n_pad,), DUMP_ROW, jnp.int32)])
    zeros_tile = jnp.zeros((ROWS_PER_TILE, DE), jnp.float32)

    partials = _sc_aggregate(feat_ext, src_p, dst_p, zeros_tile)
    b2 = b_neigh.reshape(1, D_OUT)
    return _tc_combine(feat, partials[0], partials[1], W_self, W_neigh, b2)
